# baseline (speedup 1.0000x reference)
import jax
import jax.numpy as jnp
from jax.experimental import pallas as pl
from jax.experimental.pallas import tpu as pltpu

E = 16
TILE = 1024
D = 2048


def _router_body(x_ref, w_ref, weights_ref, topw_ref, tope_ref):
    g = pl.program_id(0)
    x = x_ref[...]
    w = w_ref[...]
    lt = jax.lax.dot_general(w, x, (((0,), (1,)), ((), ())),
                             preferred_element_type=jnp.float32)
    m = jnp.max(lt, axis=0, keepdims=True)
    ex = jnp.exp(lt - m)
    s = jnp.sum(ex, axis=0, keepdims=True)
    rows = pl.ds(g * TILE, TILE)
    weights_ref[rows, :] = (ex / s).T

    row = jax.lax.broadcasted_iota(jnp.int32, lt.shape, 0)
    i1 = jnp.min(jnp.where(lt == m, row, E), axis=0, keepdims=True)
    masked = jnp.where(row == i1, -jnp.inf, lt)
    l2 = jnp.max(masked, axis=0, keepdims=True)
    i2 = jnp.min(jnp.where(masked == l2, row, E), axis=0, keepdims=True)
    e2 = jnp.exp(l2 - m)
    r = 1.0 / (1.0 + e2)
    topw_ref[rows, :] = jnp.concatenate([r, e2 * r], axis=0).T
    tope_ref[rows, :] = jnp.concatenate([i1, i2], axis=0).T


def kernel(x, W):
    B, S, _ = x.shape
    N = B * S
    x2 = x.reshape(N, D)
    grid = (N // TILE,)
    weights, topw, tope = pl.pallas_call(
        _router_body,
        grid=grid,
        in_specs=[
            pl.BlockSpec((TILE, D), lambda i: (i, 0)),
            pl.BlockSpec((D, E), lambda i: (0, 0)),
        ],
        out_specs=[
            pl.BlockSpec((N, E), lambda i: (0, 0)),
            pl.BlockSpec((N, 2), lambda i: (0, 0)),
            pl.BlockSpec((N, 2), lambda i: (0, 0)),
        ],
        out_shape=[
            jax.ShapeDtypeStruct((N, E), jnp.float32),
            jax.ShapeDtypeStruct((N, 2), jnp.float32),
            jax.ShapeDtypeStruct((N, 2), jnp.int32),
        ],
    )(x2, W)
    return (
        weights.reshape(B, S, E),
        topw.reshape(B, S, 2),
        tope.reshape(B, S, 2),
    )


# speedup vs baseline: 1.0129x; 1.0129x over previous
import jax
import jax.numpy as jnp
from jax.experimental import pallas as pl
from jax.experimental.pallas import tpu as pltpu

E = 16
TILE = 1024
D = 2048


def _router_body(x_ref, w_ref, wout_hbm, tw_hbm, te_hbm,
                 wbuf, twbuf, tebuf, sems):
    g = pl.program_id(0)
    x = x_ref[...]
    w = w_ref[...]
    lt = jax.lax.dot_general(w, x, (((0,), (1,)), ((), ())),
                             preferred_element_type=jnp.float32)
    m = jnp.max(lt, axis=0, keepdims=True)
    ex = jnp.exp(lt - m)
    s = jnp.sum(ex, axis=0, keepdims=True)
    rows = pl.ds(g * TILE, TILE)
    wbuf[rows, :] = (ex / s).T

    row = jax.lax.broadcasted_iota(jnp.int32, lt.shape, 0)
    i1 = jnp.min(jnp.where(lt == m, row, E), axis=0, keepdims=True)
    masked = jnp.where(row == i1, -jnp.inf, lt)
    l2 = jnp.max(masked, axis=0, keepdims=True)
    i2 = jnp.min(jnp.where(masked == l2, row, E), axis=0, keepdims=True)
    e2 = jnp.exp(l2 - m)
    r = 1.0 / (1.0 + e2)
    twbuf[rows, :] = jnp.concatenate([r, e2 * r], axis=0).T
    tebuf[rows, :] = jnp.concatenate([i1, i2], axis=0).T

    @pl.when(g == pl.num_programs(0) - 1)
    def _flush():
        c0 = pltpu.make_async_copy(wbuf, wout_hbm, sems.at[0])
        c1 = pltpu.make_async_copy(twbuf, tw_hbm, sems.at[1])
        c2 = pltpu.make_async_copy(tebuf, te_hbm, sems.at[2])
        c0.start()
        c1.start()
        c2.start()
        c0.wait()
        c1.wait()
        c2.wait()


def kernel(x, W):
    B, S, _ = x.shape
    N = B * S
    x2 = x.reshape(N, D)
    grid = (N // TILE,)
    weights, topw, tope = pl.pallas_call(
        _router_body,
        grid=grid,
        in_specs=[
            pl.BlockSpec((TILE, D), lambda i: (i, 0)),
            pl.BlockSpec((D, E), lambda i: (0, 0)),
        ],
        out_specs=[
            pl.BlockSpec(memory_space=pl.ANY),
            pl.BlockSpec(memory_space=pl.ANY),
            pl.BlockSpec(memory_space=pl.ANY),
        ],
        out_shape=[
            jax.ShapeDtypeStruct((N, E), jnp.float32),
            jax.ShapeDtypeStruct((N, 2), jnp.float32),
            jax.ShapeDtypeStruct((N, 2), jnp.int32),
        ],
        scratch_shapes=[
            pltpu.VMEM((N, E), jnp.float32),
            pltpu.VMEM((N, 2), jnp.float32),
            pltpu.VMEM((N, 2), jnp.int32),
            pltpu.SemaphoreType.DMA((3,)),
        ],
    )(x2, W)
    return (
        weights.reshape(B, S, E),
        topw.reshape(B, S, 2),
        tope.reshape(B, S, 2),
    )
